# baseline (speedup 1.0000x reference)
import jax
import jax.numpy as jnp
import numpy as np
from jax.experimental import pallas as pl
from jax.experimental.pallas import tpu as pltpu

EPS = np.float32(1e-07)
A = 100000
B = 128
NB = 2
R = 8
CH = 2048
TINY = np.float32(np.finfo(np.float32).tiny)

_CHUNKS = [(i * CH, CH) for i in range(A // CH)]
if A % CH:
    _CHUNKS.append(((A // CH) * CH, A % CH))


def _threefry_bits(ks0, ks1, cnt):
    rot_a = (13, 15, 26, 6)
    rot_b = (17, 29, 16, 24)
    ks2 = ks0 ^ ks1 ^ np.uint32(0x1BD11BDA)
    ks = (ks0, ks1, ks2)
    x0 = jnp.broadcast_to(ks0, cnt.shape)
    x1 = cnt + ks1
    for i in range(5):
        for r in (rot_a if i % 2 == 0 else rot_b):
            x0 = x0 + x1
            x1 = jax.lax.shift_left(x1, np.uint32(r)) | jax.lax.shift_right_logical(
                x1, np.uint32(32 - r))
            x1 = x0 ^ x1
        x0 = x0 + ks[(i + 1) % 3]
        x1 = x1 + ks[(i + 2) % 3] + np.uint32(i + 1)
    return x0 ^ x1


def _block_kernel(keys_ref, logits_ref, mask_ref, probs_ref, logp_ref, samp_ref):
    r = pl.program_id(0)
    rowv = jax.lax.broadcasted_iota(jnp.uint32, (R, 1), 0) \
        + jnp.uint32(R) * r.astype(jnp.uint32)
    rA = rowv * np.uint32(A)

    for k in range(NB):
        ks0 = keys_ref[k, 0]
        ks1 = keys_ref[k, 1]
        base = k * A

        m = jnp.full((R, 1), -jnp.inf, jnp.float32)
        for off, sz in _CHUNKS:
            m = jnp.maximum(
                m, jnp.max(logits_ref[k, :, off:off + sz], axis=1, keepdims=True))

        s = jnp.zeros((R, 1), jnp.float32)
        t = jnp.zeros((R, 1), jnp.float32)
        mm = jnp.zeros((R, 1), jnp.float32)
        for off, sz in _CHUNKS:
            e = jnp.exp(logits_ref[k, :, off:off + sz] - m)
            msk = mask_ref[:, base + off:base + off + sz]
            s = s + jnp.sum(e, axis=1, keepdims=True)
            t = t + jnp.sum(e * msk, axis=1, keepdims=True)
            mm = mm + jnp.sum(msk, axis=1, keepdims=True)
        rs = np.float32(1.0) / s
        rden = np.float32(1.0) / (t * rs + EPS * mm)

        zbest = jnp.full((R, 1), -jnp.inf, jnp.float32)
        ibest = jnp.full((R, 1), A, jnp.int32)
        for off, sz in _CHUNKS:
            e = jnp.exp(logits_ref[k, :, off:off + sz] - m)
            msk = mask_ref[:, base + off:base + off + sz]
            p = (e * rs + EPS) * msk * rden
            probs_ref[:, base + off:base + off + sz] = p
            tt = p + EPS
            logp_ref[:, base + off:base + off + sz] = jnp.log(tt)

            ci = jax.lax.broadcasted_iota(jnp.uint32, (R, sz), 1) + np.uint32(off)
            bits = _threefry_bits(ks0, ks1, rA + ci)
            fb = jax.lax.shift_right_logical(bits, np.uint32(9)) | np.uint32(0x3F800000)
            f = jax.lax.bitcast_convert_type(fb, jnp.float32) - np.float32(1.0)
            w = -jnp.log(jnp.where(f == 0.0, TINY, f))
            z = tt / w

            zc = jnp.max(z, axis=1, keepdims=True)
            ic = jnp.min(
                jnp.where(z == zc, jax.lax.bitcast_convert_type(ci, jnp.int32),
                          np.int32(A)),
                axis=1, keepdims=True)
            upd = zc > zbest
            ibest = jnp.where(upd, ic, ibest)
            zbest = jnp.where(upd, zc, zbest)
        samp_ref[:, k * 128:(k + 1) * 128] = jnp.broadcast_to(ibest, (R, 128))


def kernel(branches_logits, action_masks):
    base = jax.random.key(42)
    keys = jnp.stack([jax.random.key_data(jax.random.fold_in(base, k))
                      for k in range(NB)]).astype(jnp.uint32)

    probs, logps, samp = pl.pallas_call(
        _block_kernel,
        grid=(B // R,),
        in_specs=[
            pl.BlockSpec(memory_space=pltpu.SMEM),
            pl.BlockSpec((NB, R, A), lambda r: (0, r, 0)),
            pl.BlockSpec((R, NB * A), lambda r: (r, 0)),
        ],
        out_specs=[
            pl.BlockSpec((R, NB * A), lambda r: (r, 0)),
            pl.BlockSpec((R, NB * A), lambda r: (r, 0)),
            pl.BlockSpec((R, NB * 128), lambda r: (r, 0)),
        ],
        out_shape=[
            jax.ShapeDtypeStruct((B, NB * A), jnp.float32),
            jax.ShapeDtypeStruct((B, NB * A), jnp.float32),
            jax.ShapeDtypeStruct((B, NB * 128), jnp.int32),
        ],
        compiler_params=pltpu.CompilerParams(
            dimension_semantics=("parallel",),
            vmem_limit_bytes=112 * 1024 * 1024,
        ),
    )(keys, branches_logits, action_masks)

    output = samp[:, ::128]
    return (output, probs, logps)
